# Initial kernel scaffold; baseline (speedup 1.0000x reference)
import jax
import jax.numpy as jnp
from jax.experimental import pallas as pl


def kernel(input, target):
    raise NotImplementedError("write your pallas kernel here")


# speedup vs baseline: 19.6973x; 19.6973x over previous
import functools

import jax
import jax.numpy as jnp
from jax.experimental import pallas as pl
from jax.experimental.pallas import tpu as pltpu


def _loss_body(x_ref, t_ref, o_ref):
    x = x_ref[...]
    tgt = t_ref[0, 0, :]
    ninf = jnp.float32(-jnp.inf)

    m = jnp.max(x, axis=1, keepdims=True)
    e = jnp.exp(x - m)
    s = jnp.sum(e, axis=1, keepdims=True)

    col = jax.lax.broadcasted_iota(jnp.int32, x.shape, 1)
    xt = jnp.max(jnp.where(col == tgt[:, None], x, ninf), axis=1, keepdims=True)

    xc = jnp.where(x < xt, x, ninf)
    t1 = jnp.max(xc, axis=1, keepdims=True)
    c1 = jnp.sum(jnp.where(xc == t1, 1.0, 0.0) * jnp.where(t1 > ninf, 1.0, 0.0),
                 axis=1, keepdims=True)
    t2 = jnp.max(jnp.where(xc < t1, xc, ninf), axis=1, keepdims=True)
    t2 = jnp.where(c1 >= 2.0, t1, t2)

    logs = jnp.log(s)
    logpk = (xt - m) - logs
    pk = jnp.exp(logpk)
    p1 = jnp.exp((t1 - m) - logs)
    p2 = jnp.exp((t2 - m) - logs)
    d = 1.0 - pk + p1 + p2
    blk = jnp.sum(-(d * d) * logpk)

    @pl.when(pl.program_id(0) == 0)
    def _init():
        o_ref[0, 0] = 0.0

    o_ref[0, 0] += blk


@functools.partial(jax.jit, static_argnames=("block_rows",))
def _dual_focal_loss(x, target, block_rows=256):
    n, c = x.shape
    nb = n // block_rows
    tgt3 = target.reshape(nb, 1, block_rows)
    out = pl.pallas_call(
        _loss_body,
        grid=(nb,),
        in_specs=[
            pl.BlockSpec((block_rows, c), lambda i: (i, 0)),
            pl.BlockSpec((1, 1, block_rows), lambda i: (i, 0, 0)),
        ],
        out_specs=pl.BlockSpec(memory_space=pltpu.SMEM),
        out_shape=jax.ShapeDtypeStruct((1, 1), jnp.float32),
    )(x, tgt3)
    return out[0, 0]


def kernel(input, target):
    return _dual_focal_loss(input, target)


# speedup vs baseline: 20.0821x; 1.0195x over previous
import functools

import jax
import jax.numpy as jnp
from jax.experimental import pallas as pl
from jax.experimental.pallas import tpu as pltpu


def _loss_body(x_ref, t_ref, o_ref):
    x = x_ref[...]
    tgt = t_ref[0, 0, :]
    ninf = jnp.float32(-jnp.inf)

    m = jnp.max(x, axis=1, keepdims=True)
    e = jnp.exp(x - m)
    s = jnp.sum(e, axis=1, keepdims=True)

    col = jax.lax.broadcasted_iota(jnp.int32, x.shape, 1)
    xt = jnp.max(jnp.where(col == tgt[:, None], x, ninf), axis=1, keepdims=True)

    xc = jnp.where(x < xt, x, ninf)
    t1 = jnp.max(xc, axis=1, keepdims=True)
    lt1 = xc < t1
    c1 = jnp.sum(jnp.where(lt1, 0.0, 1.0), axis=1, keepdims=True)
    t2 = jnp.max(jnp.where(lt1, xc, ninf), axis=1, keepdims=True)
    t2 = jnp.where(c1 >= 2.0, t1, t2)

    logs = jnp.log(s)
    logpk = (xt - m) - logs
    pk = jnp.exp(logpk)
    p1 = jnp.exp((t1 - m) - logs)
    p2 = jnp.exp((t2 - m) - logs)
    d = 1.0 - pk + p1 + p2
    blk = jnp.sum(-(d * d) * logpk)

    @pl.when(pl.program_id(0) == 0)
    def _init():
        o_ref[0, 0] = 0.0

    o_ref[0, 0] += blk


@functools.partial(jax.jit, static_argnames=("block_rows",))
def _dual_focal_loss(x, target, block_rows=256):
    n, c = x.shape
    nb = n // block_rows
    tgt3 = target.reshape(nb, 1, block_rows)
    out = pl.pallas_call(
        _loss_body,
        grid=(nb,),
        in_specs=[
            pl.BlockSpec((block_rows, c), lambda i: (i, 0)),
            pl.BlockSpec((1, 1, block_rows), lambda i: (i, 0, 0)),
        ],
        out_specs=pl.BlockSpec(memory_space=pltpu.SMEM),
        out_shape=jax.ShapeDtypeStruct((1, 1), jnp.float32),
    )(x, tgt3)
    return out[0, 0]


def kernel(input, target):
    return _dual_focal_loss(input, target)


# speedup vs baseline: 23.1181x; 1.1512x over previous
import functools

import jax
import jax.numpy as jnp
from jax.experimental import pallas as pl
from jax.experimental.pallas import tpu as pltpu


def _loss_body(x_ref, t_ref, o_ref):
    x = x_ref[...]
    tgt = t_ref[0, 0, :]
    ninf = jnp.float32(-jnp.inf)

    m = jnp.max(x, axis=1, keepdims=True)
    e = jnp.exp(x - m)
    s = jnp.sum(e, axis=1, keepdims=True)

    col = jax.lax.broadcasted_iota(jnp.int32, x.shape, 1)
    xt = jnp.max(jnp.where(col == tgt[:, None], x, ninf), axis=1, keepdims=True)

    xc = jnp.where(x < xt, x, ninf)
    t1 = jnp.max(xc, axis=1, keepdims=True)
    lt1 = xc < t1
    c1 = jnp.sum(jnp.where(lt1, 0.0, 1.0), axis=1, keepdims=True)
    t2 = jnp.max(jnp.where(lt1, xc, ninf), axis=1, keepdims=True)
    t2 = jnp.where(c1 >= 2.0, t1, t2)

    logs = jnp.log(s)
    logpk = (xt - m) - logs
    pk = jnp.exp(logpk)
    p1 = jnp.exp((t1 - m) - logs)
    p2 = jnp.exp((t2 - m) - logs)
    d = 1.0 - pk + p1 + p2
    blk = jnp.sum(-(d * d) * logpk)

    @pl.when(pl.program_id(0) == 0)
    def _init():
        o_ref[0, 0] = 0.0

    o_ref[0, 0] += blk


@functools.partial(jax.jit, static_argnames=("block_rows",))
def _dual_focal_loss(x, target, block_rows=512):
    n, c = x.shape
    nb = n // block_rows
    tgt3 = target.reshape(nb, 1, block_rows)
    out = pl.pallas_call(
        _loss_body,
        grid=(nb,),
        in_specs=[
            pl.BlockSpec((block_rows, c), lambda i: (i, 0)),
            pl.BlockSpec((1, 1, block_rows), lambda i: (i, 0, 0)),
        ],
        out_specs=pl.BlockSpec(memory_space=pltpu.SMEM),
        out_shape=jax.ShapeDtypeStruct((1, 1), jnp.float32),
    )(x, tgt3)
    return out[0, 0]


def kernel(input, target):
    return _dual_focal_loss(input, target)


# speedup vs baseline: 24.1783x; 1.0459x over previous
import functools

import jax
import jax.numpy as jnp
from jax.experimental import pallas as pl
from jax.experimental.pallas import tpu as pltpu


def _loss_body(x_ref, t_ref, o_ref):
    x = x_ref[...]
    tgt = t_ref[0, 0, :]
    ninf = jnp.float32(-jnp.inf)

    m = jnp.max(x, axis=1, keepdims=True)
    e = jnp.exp(x - m)
    s = jnp.sum(e, axis=1, keepdims=True)

    col = jax.lax.broadcasted_iota(jnp.int32, x.shape, 1)
    xt = jnp.max(jnp.where(col == tgt[:, None], x, ninf), axis=1, keepdims=True)

    xc = jnp.where(x < xt, x, ninf)
    t1 = jnp.max(xc, axis=1, keepdims=True)
    lt1 = xc < t1
    c1 = jnp.sum(jnp.where(lt1, 0.0, 1.0), axis=1, keepdims=True)
    t2 = jnp.max(jnp.where(lt1, xc, ninf), axis=1, keepdims=True)
    t2 = jnp.where(c1 >= 2.0, t1, t2)

    logs = jnp.log(s)
    logpk = (xt - m) - logs
    pk = jnp.exp(logpk)
    p1 = jnp.exp((t1 - m) - logs)
    p2 = jnp.exp((t2 - m) - logs)
    d = 1.0 - pk + p1 + p2
    blk = jnp.sum(-(d * d) * logpk)

    @pl.when(pl.program_id(0) == 0)
    def _init():
        o_ref[0, 0] = 0.0

    o_ref[0, 0] += blk


@functools.partial(jax.jit, static_argnames=("block_rows",))
def _dual_focal_loss(x, target, block_rows=1024):
    n, c = x.shape
    nb = n // block_rows
    tgt3 = target.reshape(nb, 1, block_rows)
    out = pl.pallas_call(
        _loss_body,
        grid=(nb,),
        in_specs=[
            pl.BlockSpec((block_rows, c), lambda i: (i, 0)),
            pl.BlockSpec((1, 1, block_rows), lambda i: (i, 0, 0)),
        ],
        out_specs=pl.BlockSpec(memory_space=pltpu.SMEM),
        out_shape=jax.ShapeDtypeStruct((1, 1), jnp.float32),
    )(x, tgt3)
    return out[0, 0]


def kernel(input, target):
    return _dual_focal_loss(input, target)
